# baseline (speedup 1.0000x reference)
import functools

import jax
import jax.numpy as jnp
from jax import lax
from jax.experimental import pallas as pl
from jax.experimental.pallas import tpu as pltpu
from jax.experimental.pallas import tpu_sc as plsc

_L = 16


def _build(N, D, NW, ROWS, C, NCH):
    mesh = plsc.VectorSubcoreMesh(core_axis_name="c", subcore_axis_name="s")
    num_cores = mesh.num_cores

    @functools.partial(
        pl.kernel,
        out_type=jax.ShapeDtypeStruct((N, D), jnp.float32),
        mesh=mesh,
        scratch_types=[
            pltpu.VMEM((NCH, C), jnp.int32),
            pltpu.VMEM((C, D), jnp.float32),
            pltpu.VMEM((C, D), jnp.float32),
            pltpu.SemaphoreType.DMA,
            pltpu.SemaphoreType.DMA,
        ],
    )
    def run(x_hbm, idx_hbm, pe_hbm, out_hbm, idx_v, xb, pb, semx, semg):
        wid = lax.axis_index("s") * num_cores + lax.axis_index("c")
        base = wid * ROWS
        pltpu.sync_copy(idx_hbm.at[wid], idx_v)

        def chunk(g, carry):
            r0 = base + g * C
            cx = pltpu.async_copy(x_hbm.at[pl.ds(r0, C)], xb, semx)
            cg = pltpu.async_copy(pe_hbm.at[idx_v.at[g]], pb, semg)
            cx.wait()
            cg.wait()

            def col(c, carry2):
                off = c * _L
                for r in range(C):
                    xv = xb[r, pl.ds(off, _L)]
                    sv = pb[r, pl.ds(off, _L)]
                    xb[r, pl.ds(off, _L)] = xv + jnp.where(
                        xv == 0.0, jnp.zeros_like(sv), sv
                    )
                return carry2

            lax.fori_loop(0, D // _L, col, 0)
            pltpu.sync_copy(xb, out_hbm.at[pl.ds(r0, C)])
            return carry

        lax.fori_loop(0, NCH, chunk, 0)

    return run


def kernel(x, x_structure, pe):
    B, S, D = x.shape
    N = B * S
    NW = 32
    ROWS = N // NW
    C = 8
    NCH = ROWS // C
    xf = x.reshape(N, D)
    idx3 = x_structure.reshape(NW, NCH, C)
    out = _build(N, D, NW, ROWS, C, NCH)(xf, idx3, pe)
    return out.reshape(B, S, D)


# speedup vs baseline: 1.9454x; 1.9454x over previous
import functools

import jax
import jax.numpy as jnp
from jax import lax
from jax.experimental import pallas as pl
from jax.experimental.pallas import tpu as pltpu
from jax.experimental.pallas import tpu_sc as plsc

_L = 16
_NB = 4
_LEAD = 2


def _build(N, D, NW, ROWS, C, NCH):
    mesh = plsc.VectorSubcoreMesh(core_axis_name="c", subcore_axis_name="s")
    num_cores = mesh.num_cores
    NT = NCH // _NB

    @functools.partial(
        pl.kernel,
        out_type=jax.ShapeDtypeStruct((N, D), jnp.float32),
        mesh=mesh,
        scratch_types=[
            pltpu.VMEM((NCH, C), jnp.int32),
            pltpu.VMEM((_NB, C, D), jnp.float32),
            pltpu.VMEM((_NB, C, D), jnp.float32),
            pltpu.SemaphoreType.DMA((_NB,)),
            pltpu.SemaphoreType.DMA((_NB,)),
            pltpu.SemaphoreType.DMA((_NB,)),
        ],
    )
    def run(x_hbm, idx_hbm, pe_hbm, out_hbm, idx_v, xb, pb, semx, semg, semo):
        wid = lax.axis_index("s") * num_cores + lax.axis_index("c")
        base = wid * ROWS
        pltpu.sync_copy(idx_hbm.at[wid], idx_v)

        def in_copies(j, s):
            r0 = base + j * C
            return (
                pltpu.make_async_copy(x_hbm.at[pl.ds(r0, C)], xb.at[s], semx.at[s]),
                pltpu.make_async_copy(pe_hbm.at[idx_v.at[j]], pb.at[s], semg.at[s]),
            )

        def out_copy(j, s):
            r0 = base + j * C
            return pltpu.make_async_copy(xb.at[s], out_hbm.at[pl.ds(r0, C)], semo.at[s])

        def issue_in(j, s):
            for c in in_copies(j, s):
                c.start()

        def wait_in(j, s):
            for c in in_copies(j, s):
                c.wait()

        def compute(s):
            def col(c, carry):
                off = c * _L
                for r in range(C):
                    xv = xb[s, r, pl.ds(off, _L)]
                    sv = pb[s, r, pl.ds(off, _L)]
                    xb[s, r, pl.ds(off, _L)] = xv + jnp.where(
                        xv == 0.0, jnp.zeros_like(sv), sv
                    )
                return carry

            lax.fori_loop(0, D // _L, col, 0)

        def step(j, u, first, last):
            if not first:
                out_copy(j - _LEAD, (u + _LEAD) % _NB).wait()
            if not last:
                issue_in(j + _LEAD, (u + _LEAD) % _NB)
            wait_in(j, u)
            compute(u)
            out_copy(j, u).start()

        for j in range(_LEAD):
            issue_in(j, j)
        for u in range(_NB):
            step(u, u, first=(u < _LEAD), last=False)

        def outer(t, carry):
            for u in range(_NB):
                step(t * _NB + u, u, first=False, last=False)
            return carry

        lax.fori_loop(1, NT - 1, outer, 0)

        for u in range(_NB):
            j = (NT - 1) * _NB + u
            step(j, u, first=False, last=(u >= _NB - _LEAD))
        for u in range(_NB - _LEAD, _NB):
            out_copy((NT - 1) * _NB + u, u).wait()

    return run


def kernel(x, x_structure, pe):
    B, S, D = x.shape
    N = B * S
    NW = 32
    ROWS = N // NW
    C = 8
    NCH = ROWS // C
    xf = x.reshape(N, D)
    idx3 = x_structure.reshape(NW, NCH, C)
    out = _build(N, D, NW, ROWS, C, NCH)(xf, idx3, pe)
    return out.reshape(B, S, D)


# speedup vs baseline: 2.1222x; 1.0909x over previous
import functools

import jax
import jax.numpy as jnp
from jax import lax
from jax.experimental import pallas as pl
from jax.experimental.pallas import tpu as pltpu
from jax.experimental.pallas import tpu_sc as plsc

_L = 16
_NB = 4
_LEAD = 2


def _build(N, D, NW, ROWS, C, NCH):
    mesh = plsc.VectorSubcoreMesh(core_axis_name="c", subcore_axis_name="s")
    num_cores = mesh.num_cores
    NT = NCH // _NB

    @functools.partial(
        pl.kernel,
        out_type=jax.ShapeDtypeStruct((N, D), jnp.float32),
        mesh=mesh,
        scratch_types=[
            pltpu.VMEM((NCH, C), jnp.int32),
            pltpu.VMEM((_NB, C, D), jnp.float32),
            pltpu.VMEM((_NB, C, D), jnp.float32),
            pltpu.SemaphoreType.DMA((_NB,)),
            pltpu.SemaphoreType.DMA((_NB,)),
            pltpu.SemaphoreType.DMA((_NB,)),
        ],
    )
    def run(x_hbm, idx_hbm, pe_hbm, out_hbm, idx_v, xb, pb, semx, semg, semo):
        wid = lax.axis_index("s") * num_cores + lax.axis_index("c")
        base = wid * ROWS
        pltpu.sync_copy(idx_hbm.at[wid], idx_v)

        def in_copies(j, s):
            r0 = base + j * C
            return (
                pltpu.make_async_copy(x_hbm.at[pl.ds(r0, C)], xb.at[s], semx.at[s]),
                pltpu.make_async_copy(pe_hbm.at[idx_v.at[j]], pb.at[s], semg.at[s]),
            )

        def out_copy(j, s):
            r0 = base + j * C
            return pltpu.make_async_copy(xb.at[s], out_hbm.at[pl.ds(r0, C)], semo.at[s])

        def issue_in(j, s):
            for c in in_copies(j, s):
                c.start()

        def wait_in(j, s):
            for c in in_copies(j, s):
                c.wait()

        def compute(s):
            @plsc.parallel_loop(0, D // _L, unroll=4)
            def col(c):
                off = c * _L
                for r in range(C):
                    xv = xb[s, r, pl.ds(off, _L)]
                    sv = pb[s, r, pl.ds(off, _L)]
                    xb[s, r, pl.ds(off, _L)] = xv + jnp.where(
                        xv == 0.0, jnp.zeros_like(sv), sv
                    )

        def step(j, u, first, last):
            if not first:
                out_copy(j - _LEAD, (u + _LEAD) % _NB).wait()
            if not last:
                issue_in(j + _LEAD, (u + _LEAD) % _NB)
            wait_in(j, u)
            compute(u)
            out_copy(j, u).start()

        for j in range(_LEAD):
            issue_in(j, j)
        for u in range(_NB):
            step(u, u, first=(u < _LEAD), last=False)

        def outer(t, carry):
            for u in range(_NB):
                step(t * _NB + u, u, first=False, last=False)
            return carry

        lax.fori_loop(1, NT - 1, outer, 0)

        for u in range(_NB):
            j = (NT - 1) * _NB + u
            step(j, u, first=False, last=(u >= _NB - _LEAD))
        for u in range(_NB - _LEAD, _NB):
            out_copy((NT - 1) * _NB + u, u).wait()

    return run


def kernel(x, x_structure, pe):
    B, S, D = x.shape
    N = B * S
    NW = 32
    ROWS = N // NW
    C = 8
    NCH = ROWS // C
    xf = x.reshape(N, D)
    idx3 = x_structure.reshape(NW, NCH, C)
    out = _build(N, D, NW, ROWS, C, NCH)(xf, idx3, pe)
    return out.reshape(B, S, D)
